# baseline (speedup 1.0000x reference)
import jax
import jax.numpy as jnp
from jax.experimental import pallas as pl
from jax.experimental.pallas import tpu as pltpu


def _moe_body(x_ref, Wg_ref, bg_ref, W_ref, b_ref, out_ref, c_ref):
    e = pl.program_id(0)

    d = jnp.dot(
        x_ref[...].astype(jnp.bfloat16),
        W_ref[0].astype(jnp.bfloat16),
        preferred_element_type=jnp.float32,
    )

    @pl.when(e == 0)
    def _():
        logits = (
            jnp.dot(x_ref[...], Wg_ref[...], preferred_element_type=jnp.float32)
            + bg_ref[...]
        )
        m = jnp.max(logits, axis=-1, keepdims=True)
        ex = jnp.exp(logits - m)
        wts = ex / jnp.sum(ex, axis=-1, keepdims=True)
        ncols = wts.shape[-1]
        iota = jax.lax.broadcasted_iota(jnp.int32, wts.shape, 1)
        v1 = jnp.max(wts, axis=-1, keepdims=True)
        i1 = jnp.min(jnp.where(wts == v1, iota, ncols), axis=-1, keepdims=True)
        rest = jnp.where(iota == i1, -1.0, wts)
        v2 = jnp.max(rest, axis=-1, keepdims=True)
        i2 = jnp.min(jnp.where(rest == v2, iota, ncols), axis=-1, keepdims=True)
        c = jnp.where(iota == i1, v1, 0.0) + jnp.where(iota == i2, v2, 0.0)
        c_ref[...] = c
        ce0 = c[:, 0:1]
        out_ref[...] = (
            jnp.dot(c, b_ref[...], preferred_element_type=jnp.float32) + ce0 * d
        )

    @pl.when(e > 0)
    def _():
        c_all = c_ref[...]
        col = jax.lax.broadcasted_iota(jnp.int32, c_all.shape, 1)
        ce = jnp.sum(jnp.where(col == e, c_all, 0.0), axis=-1, keepdims=True)
        out_ref[...] += ce * d


def kernel(x, Wg, bg, W, b):
    T, D = x.shape
    E = W.shape[0]
    return pl.pallas_call(
        _moe_body,
        grid=(E,),
        in_specs=[
            pl.BlockSpec((T, D), lambda e: (0, 0)),
            pl.BlockSpec((D, E), lambda e: (0, 0)),
            pl.BlockSpec((1, E), lambda e: (0, 0)),
            pl.BlockSpec((1, D, D), lambda e: (e, 0, 0)),
            pl.BlockSpec((E, D), lambda e: (0, 0)),
        ],
        out_specs=pl.BlockSpec((T, D), lambda e: (0, 0)),
        out_shape=jax.ShapeDtypeStruct((T, D), jnp.float32),
        scratch_shapes=[pltpu.VMEM((T, E), jnp.float32)],
        compiler_params=pltpu.CompilerParams(
            dimension_semantics=("arbitrary",),
        ),
    )(x, Wg, bg.reshape(1, E), W, b)


# speedup vs baseline: 1.1347x; 1.1347x over previous
import jax
import jax.numpy as jnp
from jax.experimental import pallas as pl
from jax.experimental.pallas import tpu as pltpu


def _moe_body(x_ref, Wg_ref, bg_ref, W_ref, b_ref, out_ref, c_ref):
    e = pl.program_id(0)

    @pl.when(e == 0)
    def _():
        logits = (
            jnp.dot(x_ref[...], Wg_ref[...], preferred_element_type=jnp.float32)
            + bg_ref[...]
        )
        m = jnp.max(logits, axis=-1, keepdims=True)
        ex = jnp.exp(logits - m)
        wts = ex / jnp.sum(ex, axis=-1, keepdims=True)
        ncols = wts.shape[-1]
        iota = jax.lax.broadcasted_iota(jnp.int32, wts.shape, 1)
        v1 = jnp.max(wts, axis=-1, keepdims=True)
        i1 = jnp.min(jnp.where(wts == v1, iota, ncols), axis=-1, keepdims=True)
        rest = jnp.where(iota == i1, -1.0, wts)
        v2 = jnp.max(rest, axis=-1, keepdims=True)
        i2 = jnp.min(jnp.where(rest == v2, iota, ncols), axis=-1, keepdims=True)
        c = jnp.where(iota == i1, v1, 0.0) + jnp.where(iota == i2, v2, 0.0)
        c_ref[...] = c
        out_ref[...] = jnp.dot(c, b_ref[...], preferred_element_type=jnp.float32)

    c_all = c_ref[...]
    col = jax.lax.broadcasted_iota(jnp.int32, c_all.shape, 1)
    ce = jnp.sum(jnp.where(col == e, c_all, 0.0), axis=-1, keepdims=True)
    out_ref[...] += ce * jnp.dot(
        x_ref[...].astype(jnp.bfloat16),
        W_ref[0].astype(jnp.bfloat16),
        preferred_element_type=jnp.float32,
    )


def kernel(x, Wg, bg, W, b):
    T, D = x.shape
    E = W.shape[0]
    return pl.pallas_call(
        _moe_body,
        grid=(E,),
        in_specs=[
            pl.BlockSpec((T, D), lambda e: (0, 0)),
            pl.BlockSpec((D, E), lambda e: (0, 0)),
            pl.BlockSpec((1, E), lambda e: (0, 0)),
            pl.BlockSpec((1, D, D), lambda e: (e, 0, 0)),
            pl.BlockSpec((E, D), lambda e: (0, 0)),
        ],
        out_specs=pl.BlockSpec((T, D), lambda e: (0, 0)),
        out_shape=jax.ShapeDtypeStruct((T, D), jnp.float32),
        scratch_shapes=[pltpu.VMEM((T, E), jnp.float32)],
        compiler_params=pltpu.CompilerParams(
            dimension_semantics=("arbitrary",),
        ),
    )(x, Wg, bg.reshape(1, E), W, b)


# speedup vs baseline: 1.1946x; 1.0528x over previous
import jax
import jax.numpy as jnp
from jax.experimental import pallas as pl
from jax.experimental.pallas import tpu as pltpu


def _moe_body(x_ref, Wg_ref, bg_ref, W_ref, b_ref, out_ref, c_ref):
    e = pl.program_id(0)

    @pl.when(e == 0)
    def _():
        logits = jax.lax.dot_general(
            Wg_ref[...], x_ref[...],
            dimension_numbers=(((0,), (1,)), ((), ())),
            preferred_element_type=jnp.float32,
        ) + bg_ref[...]
        m = jnp.max(logits, axis=0, keepdims=True)
        ex = jnp.exp(logits - m)
        wts = ex / jnp.sum(ex, axis=0, keepdims=True)
        nrows = wts.shape[0]
        iota = jax.lax.broadcasted_iota(jnp.int32, wts.shape, 0)
        v1 = jnp.max(wts, axis=0, keepdims=True)
        i1 = jnp.min(jnp.where(wts == v1, iota, nrows), axis=0, keepdims=True)
        rest = jnp.where(iota == i1, -1.0, wts)
        v2 = jnp.max(rest, axis=0, keepdims=True)
        i2 = jnp.min(jnp.where(rest == v2, iota, nrows), axis=0, keepdims=True)
        c = (jnp.where(iota == i1, v1, 0.0) + jnp.where(iota == i2, v2, 0.0)).T
        c_ref[...] = c
        out_ref[...] = jnp.dot(c, b_ref[...], preferred_element_type=jnp.float32)

    c_all = c_ref[...]
    col = jax.lax.broadcasted_iota(jnp.int32, c_all.shape, 1)
    ce = jnp.sum(jnp.where(col == e, c_all, 0.0), axis=-1, keepdims=True)
    out_ref[...] += ce * jnp.dot(
        x_ref[...].astype(jnp.bfloat16),
        W_ref[0].astype(jnp.bfloat16),
        preferred_element_type=jnp.float32,
    )


def kernel(x, Wg, bg, W, b):
    T, D = x.shape
    E = W.shape[0]
    return pl.pallas_call(
        _moe_body,
        grid=(E,),
        in_specs=[
            pl.BlockSpec((T, D), lambda e: (0, 0)),
            pl.BlockSpec((D, E), lambda e: (0, 0)),
            pl.BlockSpec((E, 1), lambda e: (0, 0)),
            pl.BlockSpec((1, D, D), lambda e: (e, 0, 0)),
            pl.BlockSpec((E, D), lambda e: (0, 0)),
        ],
        out_specs=pl.BlockSpec((T, D), lambda e: (0, 0)),
        out_shape=jax.ShapeDtypeStruct((T, D), jnp.float32),
        scratch_shapes=[pltpu.VMEM((T, E), jnp.float32)],
        compiler_params=pltpu.CompilerParams(
            dimension_semantics=("arbitrary",),
        ),
    )(x, Wg, bg.reshape(E, 1), W, b)
